# baseline (speedup 1.0000x reference)
import functools

import jax
import jax.numpy as jnp
from jax import lax
from jax.experimental import pallas as pl
from jax.experimental.pallas import tpu as pltpu
from jax.experimental.pallas import tpu_sc as plsc

VOCAB = 1000000
DIM = 64
BATCH = 16384

_info = plsc.get_sparse_core_info()
_NC, _NS, _L = _info.num_cores, _info.num_subcores, _info.num_lanes
_NW = _NC * _NS
_BPW = BATCH // _NW
_GROUPS = _BPW // _L
_EROWS = DIM * _L // 128


def _sc_body(tw_hbm, cw_hbm, t1d_hbm, c1d_hbm, out_hbm,
             idx_t, idx_c, el_t, el_c, buf_t, buf_c, out_v, sem_t, sem_c):
    wid = lax.axis_index("s") * _NC + lax.axis_index("c")
    base = wid * _BPW

    pltpu.sync_copy(tw_hbm.at[pl.ds(base, _BPW)], idx_t)
    pltpu.sync_copy(cw_hbm.at[pl.ds(base, _BPW)], idx_c)

    def group(g, carry):
        ivt = idx_t[pl.ds(g * _L, _L)]
        ivc = idx_c[pl.ds(g * _L, _L)]
        for d in range(DIM):
            el_t[pl.ds(d * _L, _L)] = ivt + d * VOCAB
            el_c[pl.ds(d * _L, _L)] = ivc + d * VOCAB
        cp_t = pltpu.async_copy(t1d_hbm.at[el_t], buf_t, sem_t)
        cp_c = pltpu.async_copy(c1d_hbm.at[el_c], buf_c, sem_c)
        cp_t.wait()
        cp_c.wait()
        acc = buf_t[pl.ds(0, _L)] * buf_c[pl.ds(0, _L)]
        for d in range(1, DIM):
            acc = acc + (buf_t[pl.ds(d * _L, _L)]
                         * buf_c[pl.ds(d * _L, _L)])
        out_v[pl.ds(g * _L, _L)] = 1.0 / (1.0 + jnp.exp(-acc))
        return carry

    lax.fori_loop(0, _GROUPS, group, 0)
    pltpu.sync_copy(out_v, out_hbm.at[pl.ds(base, _BPW)])


@jax.jit
def _run(tw, cw, ttab, ctab):
    mesh = plsc.VectorSubcoreMesh(core_axis_name="c", subcore_axis_name="s")
    kern = functools.partial(
        pl.kernel,
        mesh=mesh,
        compiler_params=pltpu.CompilerParams(needs_layout_passes=False),
        out_type=jax.ShapeDtypeStruct((BATCH,), jnp.float32),
        scratch_types=[
            pltpu.VMEM((_BPW,), jnp.int32),
            pltpu.VMEM((_BPW,), jnp.int32),
            pltpu.VMEM((DIM * _L,), jnp.int32),
            pltpu.VMEM((DIM * _L,), jnp.int32),
            pltpu.VMEM((DIM * _L,), jnp.float32),
            pltpu.VMEM((DIM * _L,), jnp.float32),
            pltpu.VMEM((_BPW,), jnp.float32),
            pltpu.SemaphoreType.DMA,
            pltpu.SemaphoreType.DMA,
        ],
    )(_sc_body)
    return kern(tw, cw, ttab.T.reshape(-1), ctab.T.reshape(-1))


def kernel(target_word, context_word, target_table, context_table):
    tw = target_word.astype(jnp.int32)
    cw = context_word.astype(jnp.int32)
    return _run(tw, cw, target_table, context_table)


# speedup vs baseline: 8.4662x; 8.4662x over previous
import functools

import jax
import jax.numpy as jnp
from jax import lax
from jax.experimental import pallas as pl
from jax.experimental.pallas import tpu as pltpu
from jax.experimental.pallas import tpu_sc as plsc

VOCAB = 1000000
DIM = 64
BATCH = 16384

_info = plsc.get_sparse_core_info()
_NC, _NS, _L = _info.num_cores, _info.num_subcores, _info.num_lanes
_NW = _NC * _NS
_BPW = BATCH // _NW
_GROUPS = _BPW // _L
_EROWS = DIM * _L // 128


def _sc_body(tw_hbm, cw_hbm, t1d_hbm, c1d_hbm, out_hbm,
             idx_t, idx_c, el_t, el_c, buf_t, buf_c, out_v, sem_t, sem_c):
    wid = lax.axis_index("s") * _NC + lax.axis_index("c")
    base = wid * _BPW

    pltpu.sync_copy(tw_hbm.at[pl.ds(base, _BPW)], idx_t)
    pltpu.sync_copy(cw_hbm.at[pl.ds(base, _BPW)], idx_c)

    def group(g, carry):
        ivt = idx_t[pl.ds(g * _L, _L)]
        ivc = idx_c[pl.ds(g * _L, _L)]
        ivt64 = ivt * DIM
        ivc64 = ivc * DIM
        for d in range(DIM):
            el_t[pl.ds(d * _L, _L)] = ivt64 + d
            el_c[pl.ds(d * _L, _L)] = ivc64 + d
        cp_t = pltpu.async_copy(t1d_hbm.at[el_t], buf_t, sem_t)
        cp_c = pltpu.async_copy(c1d_hbm.at[el_c], buf_c, sem_c)
        cp_t.wait()
        cp_c.wait()
        acc = buf_t[pl.ds(0, _L)] * buf_c[pl.ds(0, _L)]
        for d in range(1, DIM):
            acc = acc + (buf_t[pl.ds(d * _L, _L)]
                         * buf_c[pl.ds(d * _L, _L)])
        out_v[pl.ds(g * _L, _L)] = 1.0 / (1.0 + jnp.exp(-acc))
        return carry

    lax.fori_loop(0, _GROUPS, group, 0)
    pltpu.sync_copy(out_v, out_hbm.at[pl.ds(base, _BPW)])


@jax.jit
def _run(tw, cw, ttab, ctab):
    mesh = plsc.VectorSubcoreMesh(core_axis_name="c", subcore_axis_name="s")
    kern = functools.partial(
        pl.kernel,
        mesh=mesh,
        compiler_params=pltpu.CompilerParams(needs_layout_passes=False),
        out_type=jax.ShapeDtypeStruct((BATCH,), jnp.float32),
        scratch_types=[
            pltpu.VMEM((_BPW,), jnp.int32),
            pltpu.VMEM((_BPW,), jnp.int32),
            pltpu.VMEM((DIM * _L,), jnp.int32),
            pltpu.VMEM((DIM * _L,), jnp.int32),
            pltpu.VMEM((DIM * _L,), jnp.float32),
            pltpu.VMEM((DIM * _L,), jnp.float32),
            pltpu.VMEM((_BPW,), jnp.float32),
            pltpu.SemaphoreType.DMA,
            pltpu.SemaphoreType.DMA,
        ],
    )(_sc_body)
    return kern(tw, cw, ttab.reshape(-1), ctab.reshape(-1))


def kernel(target_word, context_word, target_table, context_table):
    tw = target_word.astype(jnp.int32)
    cw = context_word.astype(jnp.int32)
    return _run(tw, cw, target_table, context_table)


# speedup vs baseline: 11.2534x; 1.3292x over previous
import functools

import jax
import jax.numpy as jnp
from jax import lax
from jax.experimental import pallas as pl
from jax.experimental.pallas import tpu as pltpu
from jax.experimental.pallas import tpu_sc as plsc

VOCAB = 1000000
DIM = 64
BATCH = 16384

_info = plsc.get_sparse_core_info()
_NC, _NS, _L = _info.num_cores, _info.num_subcores, _info.num_lanes
_NW = _NC * _NS
_BPW = BATCH // _NW
_GROUPS = _BPW // _L
_CHUNKS = DIM // _L
_TSTRIDE = _L + 1
_TILES = VOCAB // 8
_PAIRS = VOCAB // 2

_MESH = plsc.VectorSubcoreMesh(core_axis_name="c", subcore_axis_name="s")
_PARAMS = pltpu.CompilerParams(needs_layout_passes=False)


def _gather_body(tw_hbm, ttab_hbm, rows_hbm, idx_t, buf_t, rows_v, sem_t):
    wid = lax.axis_index("s") * _NC + lax.axis_index("c")
    base = wid * _BPW

    pltpu.sync_copy(tw_hbm.at[pl.ds(base, _BPW)], idx_t)
    tt3 = ttab_hbm.reshape(_TILES, 8, DIM)

    def group(g, carry):
        ivt = idx_t[pl.ds(g * _L, _L)]
        copies = [pltpu.async_copy(tt3.at[ivt[r] >> 3], buf_t.at[r], sem_t)
                  for r in range(_L)]
        for cp in copies:
            cp.wait()
        for r in range(_L):
            st = ivt[r] & 7
            for j in range(_CHUNKS):
                rows_v[pl.ds((g * _L + r) * DIM + j * _L, _L)] = (
                    buf_t[r, st, pl.ds(j * _L, _L)])
        return carry

    lax.fori_loop(0, _GROUPS, group, 0)
    pltpu.sync_copy(rows_v, rows_hbm.at[pl.ds(base * DIM, _BPW * DIM)])


def _dot_body(cw_hbm, ctab2_hbm, rows_hbm, out_hbm,
              idx_c, buf_c, rows_t, acc, out_v, sem_c, sem_r):
    wid = lax.axis_index("s") * _NC + lax.axis_index("c")
    base = wid * _BPW

    pltpu.sync_copy(cw_hbm.at[pl.ds(base, _BPW)], idx_c)
    lanes = lax.iota(jnp.int32, _L)

    def group(g, carry):
        ivc = idx_c[pl.ds(g * _L, _L)]
        cp_c = pltpu.async_copy(ctab2_hbm.at[ivc >> 1], buf_c, sem_c)
        cp_r = pltpu.async_copy(
            rows_hbm.at[pl.ds((base + g * _L) * DIM, _L * DIM)], rows_t, sem_r)
        cp_c.wait()
        cp_r.wait()
        for r in range(_L):
            hc = (ivc[r] & 1) * DIM
            p = rows_t[pl.ds(r * DIM, _L)] * buf_c[r, pl.ds(hc, _L)]
            for j in range(1, _CHUNKS):
                p = p + (rows_t[pl.ds(r * DIM + j * _L, _L)]
                         * buf_c[r, pl.ds(hc + j * _L, _L)])
            plsc.store_scatter(acc, [r * _TSTRIDE + lanes], p)
        s = plsc.load_gather(acc, [lanes * _TSTRIDE])
        for l in range(1, _L):
            s = s + plsc.load_gather(acc, [lanes * _TSTRIDE + l])
        out_v[pl.ds(g * _L, _L)] = 1.0 / (1.0 + jnp.exp(-s))
        return carry

    lax.fori_loop(0, _GROUPS, group, 0)
    pltpu.sync_copy(out_v, out_hbm.at[pl.ds(base, _BPW)])


@jax.jit
def _run(tw, cw, ttab, ctab):
    gather = functools.partial(
        pl.kernel,
        mesh=_MESH,
        compiler_params=_PARAMS,
        out_type=jax.ShapeDtypeStruct((BATCH * DIM,), jnp.float32),
        scratch_types=[
            pltpu.VMEM((_BPW,), jnp.int32),
            pltpu.VMEM((_L, 8, DIM), jnp.float32),
            pltpu.VMEM((_BPW * DIM,), jnp.float32),
            pltpu.SemaphoreType.DMA,
        ],
    )(_gather_body)
    rows_t = gather(tw, ttab)

    dot = functools.partial(
        pl.kernel,
        mesh=_MESH,
        compiler_params=_PARAMS,
        out_type=jax.ShapeDtypeStruct((BATCH,), jnp.float32),
        scratch_types=[
            pltpu.VMEM((_BPW,), jnp.int32),
            pltpu.VMEM((_L, 2 * DIM), jnp.float32),
            pltpu.VMEM((_L * DIM,), jnp.float32),
            pltpu.VMEM((_L * _TSTRIDE,), jnp.float32),
            pltpu.VMEM((_BPW,), jnp.float32),
            pltpu.SemaphoreType.DMA,
            pltpu.SemaphoreType.DMA,
        ],
    )(_dot_body)
    return dot(cw, ctab.reshape(_PAIRS, 2 * DIM), rows_t)


def kernel(target_word, context_word, target_table, context_table):
    tw = target_word.astype(jnp.int32)
    cw = context_word.astype(jnp.int32)
    return _run(tw, cw, target_table, context_table)


# speedup vs baseline: 13.5723x; 1.2061x over previous
import functools

import jax
import jax.numpy as jnp
from jax import lax
from jax.experimental import pallas as pl
from jax.experimental.pallas import tpu as pltpu
from jax.experimental.pallas import tpu_sc as plsc

VOCAB = 1000000
DIM = 64
BATCH = 16384

_info = plsc.get_sparse_core_info()
_NC, _NS, _L = _info.num_cores, _info.num_subcores, _info.num_lanes
_NW = _NC * _NS
_BPW = BATCH // _NW
_GROUPS = _BPW // _L
_CHUNKS = DIM // _L
_TSTRIDE = _L + 1
_TILES = VOCAB // 8

_MESH = plsc.VectorSubcoreMesh(core_axis_name="c", subcore_axis_name="s")
_PARAMS = pltpu.CompilerParams(needs_layout_passes=False)


def _gather_body(tw_hbm, ttab_hbm, rows_hbm, idx_t, buf_t, rows_v, sem_t):
    wid = lax.axis_index("s") * _NC + lax.axis_index("c")
    base = wid * _BPW

    pltpu.sync_copy(tw_hbm.at[pl.ds(base, _BPW)], idx_t)
    tt3 = ttab_hbm.reshape(_TILES, 8, DIM)

    def group(g, carry):
        ivt = idx_t[pl.ds(g * _L, _L)]
        copies = [pltpu.async_copy(tt3.at[ivt[r] >> 3], buf_t.at[r], sem_t)
                  for r in range(_L)]
        for cp in copies:
            cp.wait()
        for r in range(_L):
            st = ivt[r] & 7
            for j in range(_CHUNKS):
                rows_v[pl.ds((g * _L + r) * DIM + j * _L, _L)] = (
                    buf_t[r, st, pl.ds(j * _L, _L)])
        return carry

    lax.fori_loop(0, _GROUPS, group, 0)
    pltpu.sync_copy(rows_v, rows_hbm.at[pl.ds(base * DIM, _BPW * DIM)])


def _dot_body(cw_hbm, ctab_hbm, rows_hbm, out_hbm,
              idx_c, buf_c, rows_t, acc, out_v, sem_c, sem_r):
    wid = lax.axis_index("s") * _NC + lax.axis_index("c")
    base = wid * _BPW

    pltpu.sync_copy(cw_hbm.at[pl.ds(base, _BPW)], idx_c)
    ct3 = ctab_hbm.reshape(_TILES, 8, DIM)
    lanes = lax.iota(jnp.int32, _L)

    def group(g, carry):
        ivc = idx_c[pl.ds(g * _L, _L)]
        copies = [pltpu.async_copy(ct3.at[ivc[r] >> 3], buf_c.at[r], sem_c)
                  for r in range(_L)]
        cp_r = pltpu.async_copy(
            rows_hbm.at[pl.ds((base + g * _L) * DIM, _L * DIM)], rows_t, sem_r)
        for cp in copies:
            cp.wait()
        cp_r.wait()
        for r in range(_L):
            sc_ = ivc[r] & 7
            p = rows_t[pl.ds(r * DIM, _L)] * buf_c[r, sc_, pl.ds(0, _L)]
            for j in range(1, _CHUNKS):
                p = p + (rows_t[pl.ds(r * DIM + j * _L, _L)]
                         * buf_c[r, sc_, pl.ds(j * _L, _L)])
            plsc.store_scatter(acc, [r * _TSTRIDE + lanes], p)
        s = plsc.load_gather(acc, [lanes * _TSTRIDE])
        for l in range(1, _L):
            s = s + plsc.load_gather(acc, [lanes * _TSTRIDE + l])
        out_v[pl.ds(g * _L, _L)] = 1.0 / (1.0 + jnp.exp(-s))
        return carry

    lax.fori_loop(0, _GROUPS, group, 0)
    pltpu.sync_copy(out_v, out_hbm.at[pl.ds(base, _BPW)])


@jax.jit
def _run(tw, cw, ttab, ctab):
    gather = functools.partial(
        pl.kernel,
        mesh=_MESH,
        compiler_params=_PARAMS,
        out_type=jax.ShapeDtypeStruct((BATCH * DIM,), jnp.float32),
        scratch_types=[
            pltpu.VMEM((_BPW,), jnp.int32),
            pltpu.VMEM((_L, 8, DIM), jnp.float32),
            pltpu.VMEM((_BPW * DIM,), jnp.float32),
            pltpu.SemaphoreType.DMA,
        ],
    )(_gather_body)
    rows_t = gather(tw, ttab)

    dot = functools.partial(
        pl.kernel,
        mesh=_MESH,
        compiler_params=_PARAMS,
        out_type=jax.ShapeDtypeStruct((BATCH,), jnp.float32),
        scratch_types=[
            pltpu.VMEM((_BPW,), jnp.int32),
            pltpu.VMEM((_L, 8, DIM), jnp.float32),
            pltpu.VMEM((_L * DIM,), jnp.float32),
            pltpu.VMEM((_L * _TSTRIDE,), jnp.float32),
            pltpu.VMEM((_BPW,), jnp.float32),
            pltpu.SemaphoreType.DMA,
            pltpu.SemaphoreType.DMA,
        ],
    )(_dot_body)
    return dot(cw, ctab, rows_t)


def kernel(target_word, context_word, target_table, context_table):
    tw = target_word.astype(jnp.int32)
    cw = context_word.astype(jnp.int32)
    return _run(tw, cw, target_table, context_table)


# speedup vs baseline: 13.8682x; 1.0218x over previous
import functools

import jax
import jax.numpy as jnp
from jax import lax
from jax.experimental import pallas as pl
from jax.experimental.pallas import tpu as pltpu
from jax.experimental.pallas import tpu_sc as plsc

VOCAB = 1000000
DIM = 64
BATCH = 16384

_info = plsc.get_sparse_core_info()
_NC, _NS, _L = _info.num_cores, _info.num_subcores, _info.num_lanes
_NW = _NC * _NS
_BPW = BATCH // _NW
_GROUPS = _BPW // _L
_CHUNKS = DIM // _L
_TSTRIDE = _L + 1
_TILES = VOCAB // 8

_MESH = plsc.VectorSubcoreMesh(core_axis_name="c", subcore_axis_name="s")
_PARAMS = pltpu.CompilerParams(needs_layout_passes=False)


def _gather_body(tw_hbm, ttab_hbm, rows_hbm, idx_t, buf_t, rows_v, sem_t):
    wid = lax.axis_index("s") * _NC + lax.axis_index("c")
    base = wid * _BPW

    pltpu.sync_copy(tw_hbm.at[pl.ds(base, _BPW)], idx_t)
    tt3 = ttab_hbm.reshape(_TILES, 8, DIM)

    def group(g, carry):
        ivt = idx_t[pl.ds(g * _L, _L)]
        copies = [pltpu.async_copy(tt3.at[ivt[r] >> 3], buf_t.at[r], sem_t)
                  for r in range(_L)]
        for cp in copies:
            cp.wait()
        for r in range(_L):
            st = ivt[r] & 7
            for j in range(_CHUNKS):
                rows_v[pl.ds((g * _L + r) * DIM + j * _L, _L)] = (
                    buf_t[r, st, pl.ds(j * _L, _L)])
        return carry

    lax.fori_loop(0, _GROUPS, group, 0)
    pltpu.sync_copy(rows_v, rows_hbm.at[pl.ds(base * DIM, _BPW * DIM)])


def _dot_body(cw_hbm, ctab_hbm, rows_hbm, out_hbm,
              idx_c, buf_c0, buf_c1, rows_t0, rows_t1, acc, out_v,
              sem_c0, sem_c1, sem_r0, sem_r1):
    wid = lax.axis_index("s") * _NC + lax.axis_index("c")
    base = wid * _BPW

    pltpu.sync_copy(cw_hbm.at[pl.ds(base, _BPW)], idx_c)
    ct3 = ctab_hbm.reshape(_TILES, 8, DIM)
    lanes = lax.iota(jnp.int32, _L)
    bufs_c = (buf_c0, buf_c1)
    bufs_r = (rows_t0, rows_t1)
    sems_c = (sem_c0, sem_c1)
    sems_r = (sem_r0, sem_r1)

    def issue(g, b):
        ivc = idx_c[pl.ds(g * _L, _L)]
        for r in range(_L):
            pltpu.async_copy(ct3.at[ivc[r] >> 3], bufs_c[b].at[r], sems_c[b])
        pltpu.async_copy(
            rows_hbm.at[pl.ds((base + g * _L) * DIM, _L * DIM)],
            bufs_r[b], sems_r[b])

    issue(0, 0)
    issue(1, 1)

    def pair(k, carry):
        for b in range(2):
            g = 2 * k + b
            for r in range(_L):
                pltpu.make_async_copy(
                    ct3.at[0], bufs_c[b].at[r], sems_c[b]).wait()
            pltpu.make_async_copy(
                rows_hbm.at[pl.ds(0, _L * DIM)], bufs_r[b], sems_r[b]).wait()
            ivc = idx_c[pl.ds(g * _L, _L)]
            rows_t = bufs_r[b]
            buf_c = bufs_c[b]
            for r in range(_L):
                sc_ = ivc[r] & 7
                p = rows_t[pl.ds(r * DIM, _L)] * buf_c[r, sc_, pl.ds(0, _L)]
                for j in range(1, _CHUNKS):
                    p = p + (rows_t[pl.ds(r * DIM + j * _L, _L)]
                             * buf_c[r, sc_, pl.ds(j * _L, _L)])
                plsc.store_scatter(acc, [r * _TSTRIDE + lanes], p)
            s = plsc.load_gather(acc, [lanes * _TSTRIDE])
            for l in range(1, _L):
                s = s + plsc.load_gather(acc, [lanes * _TSTRIDE + l])
            out_v[pl.ds(g * _L, _L)] = 1.0 / (1.0 + jnp.exp(-s))

            @pl.when(k < _GROUPS // 2 - 1)
            def _():
                issue(g + 2, b)
        return carry

    lax.fori_loop(0, _GROUPS // 2, pair, 0)
    pltpu.sync_copy(out_v, out_hbm.at[pl.ds(base, _BPW)])


@jax.jit
def _run(tw, cw, ttab, ctab):
    gather = functools.partial(
        pl.kernel,
        mesh=_MESH,
        compiler_params=_PARAMS,
        out_type=jax.ShapeDtypeStruct((BATCH * DIM,), jnp.float32),
        scratch_types=[
            pltpu.VMEM((_BPW,), jnp.int32),
            pltpu.VMEM((_L, 8, DIM), jnp.float32),
            pltpu.VMEM((_BPW * DIM,), jnp.float32),
            pltpu.SemaphoreType.DMA,
        ],
    )(_gather_body)
    rows_t = gather(tw, ttab)

    dot = functools.partial(
        pl.kernel,
        mesh=_MESH,
        compiler_params=_PARAMS,
        out_type=jax.ShapeDtypeStruct((BATCH,), jnp.float32),
        scratch_types=[
            pltpu.VMEM((_BPW,), jnp.int32),
            pltpu.VMEM((_L, 8, DIM), jnp.float32),
            pltpu.VMEM((_L, 8, DIM), jnp.float32),
            pltpu.VMEM((_L * DIM,), jnp.float32),
            pltpu.VMEM((_L * DIM,), jnp.float32),
            pltpu.VMEM((_L * _TSTRIDE,), jnp.float32),
            pltpu.VMEM((_BPW,), jnp.float32),
            pltpu.SemaphoreType.DMA,
            pltpu.SemaphoreType.DMA,
            pltpu.SemaphoreType.DMA,
            pltpu.SemaphoreType.DMA,
        ],
    )(_dot_body)
    return dot(cw, ctab, rows_t)


def kernel(target_word, context_word, target_table, context_table):
    tw = target_word.astype(jnp.int32)
    cw = context_word.astype(jnp.int32)
    return _run(tw, cw, target_table, context_table)
